# baseline (speedup 1.0000x reference)
import functools

import jax
import jax.numpy as jnp
from jax import lax
from jax.experimental import pallas as pl
from jax.experimental.pallas import tpu as pltpu
from jax.experimental.pallas import tpu_sc as plsc

_H = 256
_W = 256
_FC = 128
_ROWS_PER_BLOCK = 8
_P = _ROWS_PER_BLOCK * _W
_D = 8

_SC_CHUNK = 128


def _pix_coords(i):
    pix = lax.broadcasted_iota(jnp.int32, (_P, 1), 0) + i * _P
    row = pix >> 8
    col = pix & (_W - 1)
    pyf = 1.0 - 2.0 * (row.astype(jnp.float32) + 0.5) / _H
    pxf = 1.0 - 2.0 * (col.astype(jnp.float32) + 0.5) / _W
    return pxf, pyf


def _raster_body(fd_ref, idx_ref, vis_ref):
    nchunks = fd_ref.shape[0]
    pxf, pyf = _pix_coords(pl.program_id(0))
    lane = lax.broadcasted_iota(jnp.int32, (_P, _FC), 1)
    inf = jnp.float32(jnp.inf)

    ones = jnp.ones((_P, 1), dtype=jnp.float32)
    zeros = jnp.zeros((_P, 5), dtype=jnp.float32)
    xmat = jnp.concatenate([pyf, pxf, ones, zeros], axis=1)

    def body(c, carry):
        run_z, run_i = carry
        cm = fd_ref[c]
        y = jnp.dot(xmat, cm, preferred_element_type=jnp.float32,
                    precision=lax.Precision.HIGHEST)
        n0 = y[:, 0:_FC]
        n1 = y[:, _FC:2 * _FC]
        n2 = y[:, 2 * _FC:3 * _FC]
        zb = y[:, 3 * _FC:4 * _FC]
        inside = jnp.minimum(jnp.minimum(n0, n1), n2) >= 0.0
        zf = jnp.where(inside, zb, inf)

        minz = jnp.min(zf, axis=1, keepdims=True)
        cand = jnp.where(zf == minz, lane, _FC)
        lanewin = jnp.min(cand, axis=1, keepdims=True)

        better = minz < run_z
        run_z = jnp.where(better, minz, run_z)
        run_i = jnp.where(better, lanewin + c * _FC, run_i)
        return run_z, run_i

    run_z0 = jnp.full((_P, 1), inf, dtype=jnp.float32)
    run_i0 = jnp.zeros((_P, 1), dtype=jnp.int32)
    run_z, run_i = lax.fori_loop(0, nchunks, body, (run_z0, run_i0))

    idx_ref[...] = run_i
    vis_ref[...] = (run_z < inf).astype(jnp.float32)


def _gather_body(coef_hbm, idx_hbm, g_hbm, idx_v, rows_v, sem):
    wid = lax.axis_index("s") * 2 + lax.axis_index("c")
    npix = (_H * _W) // 32
    nchunks = npix // _SC_CHUNK

    def chunk(ci, carry):
        base = wid * npix + ci * _SC_CHUNK
        pltpu.sync_copy(idx_hbm.at[pl.ds(base, _SC_CHUNK)], idx_v)
        pltpu.async_copy(coef_hbm.at[idx_v], rows_v, sem).wait()
        pltpu.sync_copy(rows_v, g_hbm.at[pl.ds(base, _SC_CHUNK)])
        return carry

    lax.fori_loop(0, nchunks, chunk, 0)


def _interp_body(g_ref, vis_ref, out_ref):
    pxf, pyf = _pix_coords(pl.program_id(0))
    g = g_ref[...]
    gp = g[:, 0:_D]
    gq = g[:, _D:2 * _D]
    gr = g[:, 2 * _D:3 * _D]
    vis = vis_ref[...]
    out8 = (gp * pyf + gq * pxf + gr) * vis
    out_ref[...] = jnp.concatenate(
        [out8, vis, jnp.zeros((_P, 16 - _D - 1), jnp.float32)], axis=1)


def kernel(vertices, faces, attributes):
    verts = vertices[0].astype(jnp.float32)
    f = faces[0]
    F = f.shape[0]

    fv = verts[f]
    x0, y0, z0 = fv[:, 0, 0], fv[:, 0, 1], fv[:, 0, 2]
    x1, y1, z1 = fv[:, 1, 0], fv[:, 1, 1], fv[:, 1, 2]
    x2, y2, z2 = fv[:, 2, 0], fv[:, 2, 1], fv[:, 2, 2]
    area = (x1 - x0) * (y2 - y0) - (y1 - y0) * (x2 - x0)
    valid = jnp.abs(area) > 1e-8
    den = jnp.where(valid, area, 1.0)
    s = jnp.sign(den)

    e0x, e0y = x2 - x1, y2 - y1
    e1x, e1y = x0 - x2, y0 - y2
    e2x, e2y = x1 - x0, y1 - y0
    c0 = e0y * x1 - e0x * y1
    c1 = e1y * x2 - e1x * y2
    c2 = e2y * x0 - e2x * y0

    na0, nb0 = s * e0x, -s * e0y
    na1, nb1 = s * e1x, -s * e1y
    na2, nb2 = s * e2x, -s * e2y
    nc0 = jnp.where(valid, s * c0, -1.0)
    na0 = jnp.where(valid, na0, 0.0)
    nb0 = jnp.where(valid, nb0, 0.0)
    nc1, nc2 = s * c1, s * c2
    za = (e0x * z0 + e1x * z1 + e2x * z2) / den
    zbx = -(e0y * z0 + e1y * z1 + e2y * z2) / den
    zc = (c0 * z0 + c1 * z1 + c2 * z2) / den

    Fp = ((F + _FC - 1) // _FC) * _FC
    nchunks = Fp // _FC
    pad = Fp - F

    def padf(a):
        return jnp.pad(a, (0, pad))

    def qcat(q0, q1, q2, q3, c0pad=0.0):
        rows = jnp.stack([
            jnp.pad(q0, (0, pad), constant_values=c0pad),
            padf(q1), padf(q2), padf(q3)], axis=0)
        return rows.reshape(4, nchunks, _FC).transpose(1, 0, 2).reshape(
            nchunks, 4 * _FC)

    arow = qcat(na0, na1, na2, za)
    brow = qcat(nb0, nb1, nb2, zbx)
    crow = qcat(nc0, nc1, nc2, zc, c0pad=-1.0)
    zrow = jnp.zeros_like(arow)
    fd = jnp.stack([arow, brow, crow, zrow, zrow, zrow, zrow, zrow],
                   axis=1)

    att = attributes[0].astype(jnp.float32)
    ex = jnp.stack([e0x, e1x, e2x], 1)
    ey = jnp.stack([e0y, e1y, e2y], 1)
    cc = jnp.stack([c0, c1, c2], 1)
    Pm = jnp.einsum('fk,fkd->fd', ex, att) / den[:, None]
    Qm = -jnp.einsum('fk,fkd->fd', ey, att) / den[:, None]
    Rm = jnp.einsum('fk,fkd->fd', cc, att) / den[:, None]
    coef = jnp.concatenate([Pm, Qm, Rm], axis=1)

    nblocks = (_H * _W) // _P
    idx, vis = pl.pallas_call(
        _raster_body,
        grid=(nblocks,),
        in_specs=[
            pl.BlockSpec((nchunks, 8, 4 * _FC), lambda i: (0, 0, 0)),
        ],
        out_specs=[
            pl.BlockSpec((_P, 1), lambda i: (i, 0)),
            pl.BlockSpec((_P, 1), lambda i: (i, 0)),
        ],
        out_shape=[
            jax.ShapeDtypeStruct((_H * _W, 1), jnp.int32),
            jax.ShapeDtypeStruct((_H * _W, 1), jnp.float32),
        ],
    )(fd)

    idx1 = idx.reshape(_H * _W)

    mesh = plsc.VectorSubcoreMesh(core_axis_name="c", subcore_axis_name="s")
    gathered = functools.partial(
        pl.kernel, mesh=mesh,
        out_type=jax.ShapeDtypeStruct((_H * _W, 3 * _D), jnp.float32),
        compiler_params=pltpu.CompilerParams(use_tc_tiling_on_sc=False),
        scratch_types=[
            pltpu.VMEM((_SC_CHUNK,), jnp.int32),
            pltpu.VMEM((_SC_CHUNK, 3 * _D), jnp.float32),
            pltpu.SemaphoreType.DMA,
        ],
    )(_gather_body)(coef, idx1)

    out = pl.pallas_call(
        _interp_body,
        grid=(nblocks,),
        in_specs=[
            pl.BlockSpec((_P, 3 * _D), lambda i: (i, 0)),
            pl.BlockSpec((_P, 1), lambda i: (i, 0)),
        ],
        out_specs=pl.BlockSpec((_P, 16), lambda i: (i, 0)),
        out_shape=jax.ShapeDtypeStruct((_H * _W, 16), jnp.float32),
    )(gathered, vis)

    img = out[:, 0:_D + 1].reshape(_H, _W, _D + 1).transpose(2, 0, 1)
    return img[None]


# speedup vs baseline: 2.3859x; 2.3859x over previous
import functools

import jax
import jax.numpy as jnp
from jax import lax
from jax.experimental import pallas as pl
from jax.experimental.pallas import tpu as pltpu
from jax.experimental.pallas import tpu_sc as plsc

_H = 256
_W = 256
_FC = 128
_ROWS_PER_BLOCK = 8
_P = _ROWS_PER_BLOCK * _W
_D = 8

_SC_CHUNK = 128


def _pix_coords(i):
    pix = lax.broadcasted_iota(jnp.int32, (_P, 1), 0) + i * _P
    row = pix >> 8
    col = pix & (_W - 1)
    pyf = 1.0 - 2.0 * (row.astype(jnp.float32) + 0.5) / _H
    pxf = 1.0 - 2.0 * (col.astype(jnp.float32) + 0.5) / _W
    return pxf, pyf


def _raster_body(climit_ref, fd_ref, idx_ref, vis_ref):
    i = pl.program_id(0)
    pxf, pyf = _pix_coords(i)
    lane = lax.broadcasted_iota(jnp.int32, (_P, _FC), 1)
    inf = jnp.float32(jnp.inf)

    def body(c, carry):
        run_z, run_i = carry
        fd = fd_ref[c]
        na0, nb0, nc0 = fd[0:1], fd[1:2], fd[2:3]
        na1, nb1, nc1 = fd[3:4], fd[4:5], fd[5:6]
        absden, u0, u1, z2 = fd[6:7], fd[7:8], fd[8:9], fd[9:10]

        n0 = na0 * pyf + nb0 * pxf + nc0
        n1 = na1 * pyf + nb1 * pxf + nc1
        n2 = absden - (n0 + n1)
        inside = jnp.minimum(jnp.minimum(n0, n1), n2) >= 0.0
        zb = n0 * u0 + n1 * u1 + z2
        zf = jnp.where(inside, zb, inf)

        minz = jnp.min(zf, axis=1, keepdims=True)
        cand = jnp.where(zf == minz, lane, _FC)
        lanewin = jnp.min(cand, axis=1, keepdims=True)

        better = minz < run_z
        run_z = jnp.where(better, minz, run_z)
        run_i = jnp.where(better, lanewin + c * _FC, run_i)
        return run_z, run_i

    run_z0 = jnp.full((_P, 1), inf, dtype=jnp.float32)
    run_i0 = jnp.zeros((_P, 1), dtype=jnp.int32)
    run_z, run_i = lax.fori_loop(0, climit_ref[i], body, (run_z0, run_i0))

    idx_ref[...] = run_i
    vis_ref[...] = (run_z < inf).astype(jnp.float32)


def _gather_body(coef_hbm, idx_hbm, g_hbm, idx_v, rows_v, sem):
    wid = lax.axis_index("s") * 2 + lax.axis_index("c")
    npix = (_H * _W) // 32
    nchunks = npix // _SC_CHUNK

    def chunk(ci, carry):
        base = wid * npix + ci * _SC_CHUNK
        pltpu.sync_copy(idx_hbm.at[pl.ds(base, _SC_CHUNK)], idx_v)
        pltpu.async_copy(coef_hbm.at[idx_v], rows_v, sem).wait()
        pltpu.sync_copy(rows_v, g_hbm.at[pl.ds(base, _SC_CHUNK)])
        return carry

    lax.fori_loop(0, nchunks, chunk, 0)


def _interp_body(g_ref, vis_ref, out_ref):
    pxf, pyf = _pix_coords(pl.program_id(0))
    g = g_ref[...]
    gp = g[:, 0:_D]
    gq = g[:, _D:2 * _D]
    gr = g[:, 2 * _D:3 * _D]
    vis = vis_ref[...]
    out8 = (gp * pyf + gq * pxf + gr) * vis
    out_ref[...] = jnp.concatenate(
        [out8, vis, jnp.zeros((_P, 16 - _D - 1), jnp.float32)], axis=1)


def kernel(vertices, faces, attributes):
    verts = vertices[0].astype(jnp.float32)
    f = faces[0]
    F = f.shape[0]

    fv = verts[f]
    x0, y0, z0 = fv[:, 0, 0], fv[:, 0, 1], fv[:, 0, 2]
    x1, y1, z1 = fv[:, 1, 0], fv[:, 1, 1], fv[:, 1, 2]
    x2, y2, z2 = fv[:, 2, 0], fv[:, 2, 1], fv[:, 2, 2]

    ymin = jnp.minimum(jnp.minimum(y0, y1), y2)
    order = jnp.argsort(ymin, stable=True)
    ymin_s = ymin[order]
    fvs = fv[order]
    x0, y0, z0 = fvs[:, 0, 0], fvs[:, 0, 1], fvs[:, 0, 2]
    x1, y1, z1 = fvs[:, 1, 0], fvs[:, 1, 1], fvs[:, 1, 2]
    x2, y2, z2 = fvs[:, 2, 0], fvs[:, 2, 1], fvs[:, 2, 2]

    area = (x1 - x0) * (y2 - y0) - (y1 - y0) * (x2 - x0)
    valid = jnp.abs(area) > 1e-8
    den = jnp.where(valid, area, 1.0)
    s = jnp.sign(den)

    e0x, e0y = x2 - x1, y2 - y1
    e1x, e1y = x0 - x2, y0 - y2
    c0 = e0y * x1 - e0x * y1
    c1 = e1y * x2 - e1x * y2

    na0, nb0 = s * e0x, -s * e0y
    na1, nb1 = s * e1x, -s * e1y
    nc0 = jnp.where(valid, s * c0, -1.0)
    na0 = jnp.where(valid, na0, 0.0)
    nb0 = jnp.where(valid, nb0, 0.0)
    nc1 = s * c1
    absden = jnp.abs(den)
    u0 = (z0 - z2) / absden
    u1 = (z1 - z2) / absden

    Fp = ((F + _FC - 1) // _FC) * _FC
    nchunks = Fp // _FC
    pad = Fp - F

    def padf(a):
        return jnp.pad(a, (0, pad))

    zero = jnp.zeros((Fp,), jnp.float32)
    fd = jnp.stack([
        padf(na0), padf(nb0), jnp.pad(nc0, (0, pad), constant_values=-1.0),
        padf(na1), padf(nb1), padf(nc1),
        padf(absden), padf(u0), padf(u1), padf(z2),
        zero, zero, zero, zero, zero, zero,
    ], axis=0)
    fd = fd.reshape(16, nchunks, _FC).transpose(1, 0, 2)

    nblocks = (_H * _W) // _P
    r0 = jnp.arange(nblocks, dtype=jnp.float32) * _ROWS_PER_BLOCK
    yhi = 1.0 - 2.0 * (r0 + 0.5) / _H
    counts = jnp.searchsorted(ymin_s, yhi, side='right')
    climit = ((counts + _FC - 1) // _FC).astype(jnp.int32)

    att = attributes[0].astype(jnp.float32)[order]
    e2x, e2y = x1 - x0, y1 - y0
    c2 = e2y * x0 - e2x * y0
    ex = jnp.stack([e0x, e1x, e2x], 1)
    ey = jnp.stack([e0y, e1y, e2y], 1)
    cc = jnp.stack([c0, c1, c2], 1)
    Pm = jnp.einsum('fk,fkd->fd', ex, att) / den[:, None]
    Qm = -jnp.einsum('fk,fkd->fd', ey, att) / den[:, None]
    Rm = jnp.einsum('fk,fkd->fd', cc, att) / den[:, None]
    coef = jnp.concatenate([Pm, Qm, Rm], axis=1)

    idx, vis = pl.pallas_call(
        _raster_body,
        grid_spec=pltpu.PrefetchScalarGridSpec(
            num_scalar_prefetch=1,
            grid=(nblocks,),
            in_specs=[
                pl.BlockSpec((nchunks, 16, _FC), lambda i, c: (0, 0, 0)),
            ],
            out_specs=[
                pl.BlockSpec((_P, 1), lambda i, c: (i, 0)),
                pl.BlockSpec((_P, 1), lambda i, c: (i, 0)),
            ],
        ),
        out_shape=[
            jax.ShapeDtypeStruct((_H * _W, 1), jnp.int32),
            jax.ShapeDtypeStruct((_H * _W, 1), jnp.float32),
        ],
    )(climit, fd)

    idx1 = idx.reshape(_H * _W)

    mesh = plsc.VectorSubcoreMesh(core_axis_name="c", subcore_axis_name="s")
    gathered = functools.partial(
        pl.kernel, mesh=mesh,
        out_type=jax.ShapeDtypeStruct((_H * _W, 3 * _D), jnp.float32),
        compiler_params=pltpu.CompilerParams(use_tc_tiling_on_sc=False),
        scratch_types=[
            pltpu.VMEM((_SC_CHUNK,), jnp.int32),
            pltpu.VMEM((_SC_CHUNK, 3 * _D), jnp.float32),
            pltpu.SemaphoreType.DMA,
        ],
    )(_gather_body)(coef, idx1)

    out = pl.pallas_call(
        _interp_body,
        grid=(nblocks,),
        in_specs=[
            pl.BlockSpec((_P, 3 * _D), lambda i: (i, 0)),
            pl.BlockSpec((_P, 1), lambda i: (i, 0)),
        ],
        out_specs=pl.BlockSpec((_P, 16), lambda i: (i, 0)),
        out_shape=jax.ShapeDtypeStruct((_H * _W, 16), jnp.float32),
    )(gathered, vis)

    img = out[:, 0:_D + 1].reshape(_H, _W, _D + 1).transpose(2, 0, 1)
    return img[None]


# speedup vs baseline: 3.4133x; 1.4306x over previous
import functools

import jax
import jax.numpy as jnp
from jax import lax
from jax.experimental import pallas as pl
from jax.experimental.pallas import tpu as pltpu
from jax.experimental.pallas import tpu_sc as plsc

_H = 256
_W = 256
_FC = 512
_ROWS_PER_BLOCK = 8
_P = _ROWS_PER_BLOCK * _W
_D = 8

_SC_CHUNK = 128


def _pix_coords(i):
    pix = lax.broadcasted_iota(jnp.int32, (_P, 1), 0) + i * _P
    row = pix >> 8
    col = pix & (_W - 1)
    pyf = 1.0 - 2.0 * (row.astype(jnp.float32) + 0.5) / _H
    pxf = 1.0 - 2.0 * (col.astype(jnp.float32) + 0.5) / _W
    return pxf, pyf


def _raster_body(climit_ref, fd_ref, idx_ref, vis_ref):
    i = pl.program_id(0)
    pxf, pyf = _pix_coords(i)
    lane = lax.broadcasted_iota(jnp.int32, (_P, _FC), 1)
    inf = jnp.float32(jnp.inf)

    def body(c, carry):
        run_z, run_i = carry
        fd = fd_ref[c]
        na0, nb0, nc0 = fd[0:1], fd[1:2], fd[2:3]
        na1, nb1, nc1 = fd[3:4], fd[4:5], fd[5:6]
        absden, u0, u1, z2 = fd[6:7], fd[7:8], fd[8:9], fd[9:10]

        n0 = na0 * pyf + nb0 * pxf + nc0
        n1 = na1 * pyf + nb1 * pxf + nc1
        n2 = absden - (n0 + n1)
        inside = jnp.minimum(jnp.minimum(n0, n1), n2) >= 0.0
        zb = n0 * u0 + n1 * u1 + z2
        zf = jnp.where(inside, zb, inf)

        minz = jnp.min(zf, axis=1, keepdims=True)
        cand = jnp.where(zf == minz, lane, _FC)
        lanewin = jnp.min(cand, axis=1, keepdims=True)

        better = minz < run_z
        run_z = jnp.where(better, minz, run_z)
        run_i = jnp.where(better, lanewin + c * _FC, run_i)
        return run_z, run_i

    run_z0 = jnp.full((_P, 1), inf, dtype=jnp.float32)
    run_i0 = jnp.zeros((_P, 1), dtype=jnp.int32)
    run_z, run_i = lax.fori_loop(0, climit_ref[i], body, (run_z0, run_i0))

    idx_ref[...] = run_i
    vis_ref[...] = (run_z < inf).astype(jnp.float32)


def _gather_body(coef_hbm, idx_hbm, g_hbm, idx_v, rows_v, sem):
    wid = lax.axis_index("s") * 2 + lax.axis_index("c")
    npix = (_H * _W) // 32
    nchunks = npix // _SC_CHUNK

    def chunk(ci, carry):
        base = wid * npix + ci * _SC_CHUNK
        pltpu.sync_copy(idx_hbm.at[pl.ds(base, _SC_CHUNK)], idx_v)
        pltpu.async_copy(coef_hbm.at[idx_v], rows_v, sem).wait()
        pltpu.sync_copy(rows_v, g_hbm.at[pl.ds(base, _SC_CHUNK)])
        return carry

    lax.fori_loop(0, nchunks, chunk, 0)


def _interp_body(g_ref, vis_ref, out_ref):
    pxf, pyf = _pix_coords(pl.program_id(0))
    g = g_ref[...]
    gp = g[:, 0:_D]
    gq = g[:, _D:2 * _D]
    gr = g[:, 2 * _D:3 * _D]
    vis = vis_ref[...]
    out8 = (gp * pyf + gq * pxf + gr) * vis
    out_ref[...] = jnp.concatenate(
        [out8, vis, jnp.zeros((_P, 16 - _D - 1), jnp.float32)], axis=1)


def kernel(vertices, faces, attributes):
    verts = vertices[0].astype(jnp.float32)
    f = faces[0]
    F = f.shape[0]

    fv = verts[f]
    x0, y0, z0 = fv[:, 0, 0], fv[:, 0, 1], fv[:, 0, 2]
    x1, y1, z1 = fv[:, 1, 0], fv[:, 1, 1], fv[:, 1, 2]
    x2, y2, z2 = fv[:, 2, 0], fv[:, 2, 1], fv[:, 2, 2]

    ymin = jnp.minimum(jnp.minimum(y0, y1), y2)
    order = jnp.argsort(ymin, stable=True)
    ymin_s = ymin[order]
    fvs = fv[order]
    x0, y0, z0 = fvs[:, 0, 0], fvs[:, 0, 1], fvs[:, 0, 2]
    x1, y1, z1 = fvs[:, 1, 0], fvs[:, 1, 1], fvs[:, 1, 2]
    x2, y2, z2 = fvs[:, 2, 0], fvs[:, 2, 1], fvs[:, 2, 2]

    area = (x1 - x0) * (y2 - y0) - (y1 - y0) * (x2 - x0)
    valid = jnp.abs(area) > 1e-8
    den = jnp.where(valid, area, 1.0)
    s = jnp.sign(den)

    e0x, e0y = x2 - x1, y2 - y1
    e1x, e1y = x0 - x2, y0 - y2
    c0 = e0y * x1 - e0x * y1
    c1 = e1y * x2 - e1x * y2

    na0, nb0 = s * e0x, -s * e0y
    na1, nb1 = s * e1x, -s * e1y
    nc0 = jnp.where(valid, s * c0, -1.0)
    na0 = jnp.where(valid, na0, 0.0)
    nb0 = jnp.where(valid, nb0, 0.0)
    nc1 = s * c1
    absden = jnp.abs(den)
    u0 = (z0 - z2) / absden
    u1 = (z1 - z2) / absden

    Fp = ((F + _FC - 1) // _FC) * _FC
    nchunks = Fp // _FC
    pad = Fp - F

    def padf(a):
        return jnp.pad(a, (0, pad))

    zero = jnp.zeros((Fp,), jnp.float32)
    fd = jnp.stack([
        padf(na0), padf(nb0), jnp.pad(nc0, (0, pad), constant_values=-1.0),
        padf(na1), padf(nb1), padf(nc1),
        padf(absden), padf(u0), padf(u1), padf(z2),
        zero, zero, zero, zero, zero, zero,
    ], axis=0)
    fd = fd.reshape(16, nchunks, _FC).transpose(1, 0, 2)

    nblocks = (_H * _W) // _P
    r0 = jnp.arange(nblocks, dtype=jnp.float32) * _ROWS_PER_BLOCK
    yhi = 1.0 - 2.0 * (r0 + 0.5) / _H
    counts = jnp.searchsorted(ymin_s, yhi, side='right')
    climit = ((counts + _FC - 1) // _FC).astype(jnp.int32)

    att = attributes[0].astype(jnp.float32)[order]
    e2x, e2y = x1 - x0, y1 - y0
    c2 = e2y * x0 - e2x * y0
    ex = jnp.stack([e0x, e1x, e2x], 1)
    ey = jnp.stack([e0y, e1y, e2y], 1)
    cc = jnp.stack([c0, c1, c2], 1)
    Pm = jnp.einsum('fk,fkd->fd', ex, att) / den[:, None]
    Qm = -jnp.einsum('fk,fkd->fd', ey, att) / den[:, None]
    Rm = jnp.einsum('fk,fkd->fd', cc, att) / den[:, None]
    coef = jnp.concatenate([Pm, Qm, Rm], axis=1)

    idx, vis = pl.pallas_call(
        _raster_body,
        grid_spec=pltpu.PrefetchScalarGridSpec(
            num_scalar_prefetch=1,
            grid=(nblocks,),
            in_specs=[
                pl.BlockSpec((nchunks, 16, _FC), lambda i, c: (0, 0, 0)),
            ],
            out_specs=[
                pl.BlockSpec((_P, 1), lambda i, c: (i, 0)),
                pl.BlockSpec((_P, 1), lambda i, c: (i, 0)),
            ],
        ),
        out_shape=[
            jax.ShapeDtypeStruct((_H * _W, 1), jnp.int32),
            jax.ShapeDtypeStruct((_H * _W, 1), jnp.float32),
        ],
    )(climit, fd)

    idx1 = idx.reshape(_H * _W)

    mesh = plsc.VectorSubcoreMesh(core_axis_name="c", subcore_axis_name="s")
    gathered = functools.partial(
        pl.kernel, mesh=mesh,
        out_type=jax.ShapeDtypeStruct((_H * _W, 3 * _D), jnp.float32),
        compiler_params=pltpu.CompilerParams(use_tc_tiling_on_sc=False),
        scratch_types=[
            pltpu.VMEM((_SC_CHUNK,), jnp.int32),
            pltpu.VMEM((_SC_CHUNK, 3 * _D), jnp.float32),
            pltpu.SemaphoreType.DMA,
        ],
    )(_gather_body)(coef, idx1)

    out = pl.pallas_call(
        _interp_body,
        grid=(nblocks,),
        in_specs=[
            pl.BlockSpec((_P, 3 * _D), lambda i: (i, 0)),
            pl.BlockSpec((_P, 1), lambda i: (i, 0)),
        ],
        out_specs=pl.BlockSpec((_P, 16), lambda i: (i, 0)),
        out_shape=jax.ShapeDtypeStruct((_H * _W, 16), jnp.float32),
    )(gathered, vis)

    img = out[:, 0:_D + 1].reshape(_H, _W, _D + 1).transpose(2, 0, 1)
    return img[None]


# speedup vs baseline: 3.4877x; 1.0218x over previous
import functools

import jax
import jax.numpy as jnp
from jax import lax
from jax.experimental import pallas as pl
from jax.experimental.pallas import tpu as pltpu
from jax.experimental.pallas import tpu_sc as plsc

_H = 256
_W = 256
_FC = 1024
_ROWS_PER_BLOCK = 4
_P = _ROWS_PER_BLOCK * _W
_D = 8

_SC_CHUNK = 128


def _pix_coords(i):
    pix = lax.broadcasted_iota(jnp.int32, (_P, 1), 0) + i * _P
    row = pix >> 8
    col = pix & (_W - 1)
    pyf = 1.0 - 2.0 * (row.astype(jnp.float32) + 0.5) / _H
    pxf = 1.0 - 2.0 * (col.astype(jnp.float32) + 0.5) / _W
    return pxf, pyf


def _raster_body(climit_ref, fd_ref, idx_ref, vis_ref):
    i = pl.program_id(0)
    pxf, pyf = _pix_coords(i)
    lane = lax.broadcasted_iota(jnp.int32, (_P, _FC), 1)
    inf = jnp.float32(jnp.inf)

    def body(c, carry):
        run_z, run_i = carry
        fd = fd_ref[c]
        na0, nb0, nc0 = fd[0:1], fd[1:2], fd[2:3]
        na1, nb1, nc1 = fd[3:4], fd[4:5], fd[5:6]
        absden, u0, u1, z2 = fd[6:7], fd[7:8], fd[8:9], fd[9:10]

        n0 = na0 * pyf + nb0 * pxf + nc0
        n1 = na1 * pyf + nb1 * pxf + nc1
        n2 = absden - (n0 + n1)
        inside = jnp.minimum(jnp.minimum(n0, n1), n2) >= 0.0
        zb = n0 * u0 + n1 * u1 + z2
        zf = jnp.where(inside, zb, inf)

        minz = jnp.min(zf, axis=1, keepdims=True)
        cand = jnp.where(zf == minz, lane, _FC)
        lanewin = jnp.min(cand, axis=1, keepdims=True)

        better = minz < run_z
        run_z = jnp.where(better, minz, run_z)
        run_i = jnp.where(better, lanewin + c * _FC, run_i)
        return run_z, run_i

    run_z0 = jnp.full((_P, 1), inf, dtype=jnp.float32)
    run_i0 = jnp.zeros((_P, 1), dtype=jnp.int32)
    run_z, run_i = lax.fori_loop(0, climit_ref[i], body, (run_z0, run_i0))

    idx_ref[...] = run_i
    vis_ref[...] = (run_z < inf).astype(jnp.float32)


def _gather_body(coef_hbm, idx_hbm, g_hbm, idx_v, rows_v, sem):
    wid = lax.axis_index("s") * 2 + lax.axis_index("c")
    npix = (_H * _W) // 32
    nchunks = npix // _SC_CHUNK

    def chunk(ci, carry):
        base = wid * npix + ci * _SC_CHUNK
        pltpu.sync_copy(idx_hbm.at[pl.ds(base, _SC_CHUNK)], idx_v)
        pltpu.async_copy(coef_hbm.at[idx_v], rows_v, sem).wait()
        pltpu.sync_copy(rows_v, g_hbm.at[pl.ds(base, _SC_CHUNK)])
        return carry

    lax.fori_loop(0, nchunks, chunk, 0)


def _interp_body(g_ref, vis_ref, out_ref):
    pxf, pyf = _pix_coords(pl.program_id(0))
    g = g_ref[...]
    gp = g[:, 0:_D]
    gq = g[:, _D:2 * _D]
    gr = g[:, 2 * _D:3 * _D]
    vis = vis_ref[...]
    out8 = (gp * pyf + gq * pxf + gr) * vis
    out_ref[...] = jnp.concatenate(
        [out8, vis, jnp.zeros((_P, 16 - _D - 1), jnp.float32)], axis=1)


def kernel(vertices, faces, attributes):
    verts = vertices[0].astype(jnp.float32)
    f = faces[0]
    F = f.shape[0]

    fv = verts[f]
    x0, y0, z0 = fv[:, 0, 0], fv[:, 0, 1], fv[:, 0, 2]
    x1, y1, z1 = fv[:, 1, 0], fv[:, 1, 1], fv[:, 1, 2]
    x2, y2, z2 = fv[:, 2, 0], fv[:, 2, 1], fv[:, 2, 2]

    ymin = jnp.minimum(jnp.minimum(y0, y1), y2)
    order = jnp.argsort(ymin, stable=True)
    ymin_s = ymin[order]
    fvs = fv[order]
    x0, y0, z0 = fvs[:, 0, 0], fvs[:, 0, 1], fvs[:, 0, 2]
    x1, y1, z1 = fvs[:, 1, 0], fvs[:, 1, 1], fvs[:, 1, 2]
    x2, y2, z2 = fvs[:, 2, 0], fvs[:, 2, 1], fvs[:, 2, 2]

    area = (x1 - x0) * (y2 - y0) - (y1 - y0) * (x2 - x0)
    valid = jnp.abs(area) > 1e-8
    den = jnp.where(valid, area, 1.0)
    s = jnp.sign(den)

    e0x, e0y = x2 - x1, y2 - y1
    e1x, e1y = x0 - x2, y0 - y2
    c0 = e0y * x1 - e0x * y1
    c1 = e1y * x2 - e1x * y2

    na0, nb0 = s * e0x, -s * e0y
    na1, nb1 = s * e1x, -s * e1y
    nc0 = jnp.where(valid, s * c0, -1.0)
    na0 = jnp.where(valid, na0, 0.0)
    nb0 = jnp.where(valid, nb0, 0.0)
    nc1 = s * c1
    absden = jnp.abs(den)
    u0 = (z0 - z2) / absden
    u1 = (z1 - z2) / absden

    Fp = ((F + _FC - 1) // _FC) * _FC
    nchunks = Fp // _FC
    pad = Fp - F

    def padf(a):
        return jnp.pad(a, (0, pad))

    zero = jnp.zeros((Fp,), jnp.float32)
    fd = jnp.stack([
        padf(na0), padf(nb0), jnp.pad(nc0, (0, pad), constant_values=-1.0),
        padf(na1), padf(nb1), padf(nc1),
        padf(absden), padf(u0), padf(u1), padf(z2),
        zero, zero, zero, zero, zero, zero,
    ], axis=0)
    fd = fd.reshape(16, nchunks, _FC).transpose(1, 0, 2)

    nblocks = (_H * _W) // _P
    r0 = jnp.arange(nblocks, dtype=jnp.float32) * _ROWS_PER_BLOCK
    yhi = 1.0 - 2.0 * (r0 + 0.5) / _H
    counts = jnp.searchsorted(ymin_s, yhi, side='right')
    climit = ((counts + _FC - 1) // _FC).astype(jnp.int32)

    att = attributes[0].astype(jnp.float32)[order]
    e2x, e2y = x1 - x0, y1 - y0
    c2 = e2y * x0 - e2x * y0
    ex = jnp.stack([e0x, e1x, e2x], 1)
    ey = jnp.stack([e0y, e1y, e2y], 1)
    cc = jnp.stack([c0, c1, c2], 1)
    Pm = jnp.einsum('fk,fkd->fd', ex, att) / den[:, None]
    Qm = -jnp.einsum('fk,fkd->fd', ey, att) / den[:, None]
    Rm = jnp.einsum('fk,fkd->fd', cc, att) / den[:, None]
    coef = jnp.concatenate([Pm, Qm, Rm], axis=1)

    idx, vis = pl.pallas_call(
        _raster_body,
        grid_spec=pltpu.PrefetchScalarGridSpec(
            num_scalar_prefetch=1,
            grid=(nblocks,),
            in_specs=[
                pl.BlockSpec((nchunks, 16, _FC), lambda i, c: (0, 0, 0)),
            ],
            out_specs=[
                pl.BlockSpec((_P, 1), lambda i, c: (i, 0)),
                pl.BlockSpec((_P, 1), lambda i, c: (i, 0)),
            ],
        ),
        out_shape=[
            jax.ShapeDtypeStruct((_H * _W, 1), jnp.int32),
            jax.ShapeDtypeStruct((_H * _W, 1), jnp.float32),
        ],
    )(climit, fd)

    idx1 = idx.reshape(_H * _W)

    mesh = plsc.VectorSubcoreMesh(core_axis_name="c", subcore_axis_name="s")
    gathered = functools.partial(
        pl.kernel, mesh=mesh,
        out_type=jax.ShapeDtypeStruct((_H * _W, 3 * _D), jnp.float32),
        compiler_params=pltpu.CompilerParams(use_tc_tiling_on_sc=False),
        scratch_types=[
            pltpu.VMEM((_SC_CHUNK,), jnp.int32),
            pltpu.VMEM((_SC_CHUNK, 3 * _D), jnp.float32),
            pltpu.SemaphoreType.DMA,
        ],
    )(_gather_body)(coef, idx1)

    out = pl.pallas_call(
        _interp_body,
        grid=(nblocks,),
        in_specs=[
            pl.BlockSpec((_P, 3 * _D), lambda i: (i, 0)),
            pl.BlockSpec((_P, 1), lambda i: (i, 0)),
        ],
        out_specs=pl.BlockSpec((_P, 16), lambda i: (i, 0)),
        out_shape=jax.ShapeDtypeStruct((_H * _W, 16), jnp.float32),
    )(gathered, vis)

    img = out[:, 0:_D + 1].reshape(_H, _W, _D + 1).transpose(2, 0, 1)
    return img[None]


# speedup vs baseline: 3.4916x; 1.0011x over previous
import functools

import jax
import jax.numpy as jnp
from jax import lax
from jax.experimental import pallas as pl
from jax.experimental.pallas import tpu as pltpu
from jax.experimental.pallas import tpu_sc as plsc

_H = 256
_W = 256
_FC = 1024
_ROWS_PER_BLOCK = 8
_P = _ROWS_PER_BLOCK * _W
_D = 8

_SC_CHUNK = 128


def _pix_coords(i):
    pix = lax.broadcasted_iota(jnp.int32, (_P, 1), 0) + i * _P
    row = pix >> 8
    col = pix & (_W - 1)
    pyf = 1.0 - 2.0 * (row.astype(jnp.float32) + 0.5) / _H
    pxf = 1.0 - 2.0 * (col.astype(jnp.float32) + 0.5) / _W
    return pxf, pyf


def _raster_body(climit_ref, fd_ref, idx_ref, vis_ref):
    i = pl.program_id(0)
    pxf, pyf = _pix_coords(i)
    lane = lax.broadcasted_iota(jnp.int32, (_P, _FC), 1)
    inf = jnp.float32(jnp.inf)

    def body(c, carry):
        run_z, run_i = carry
        fd = fd_ref[c]
        na0, nb0, nc0 = fd[0:1], fd[1:2], fd[2:3]
        na1, nb1, nc1 = fd[3:4], fd[4:5], fd[5:6]
        absden, u0, u1, z2 = fd[6:7], fd[7:8], fd[8:9], fd[9:10]

        n0 = na0 * pyf + nb0 * pxf + nc0
        n1 = na1 * pyf + nb1 * pxf + nc1
        n2 = absden - (n0 + n1)
        inside = jnp.minimum(jnp.minimum(n0, n1), n2) >= 0.0
        zb = n0 * u0 + n1 * u1 + z2
        zf = jnp.where(inside, zb, inf)

        minz = jnp.min(zf, axis=1, keepdims=True)
        cand = jnp.where(zf == minz, lane, _FC)
        lanewin = jnp.min(cand, axis=1, keepdims=True)

        better = minz < run_z
        run_z = jnp.where(better, minz, run_z)
        run_i = jnp.where(better, lanewin + c * _FC, run_i)
        return run_z, run_i

    run_z0 = jnp.full((_P, 1), inf, dtype=jnp.float32)
    run_i0 = jnp.zeros((_P, 1), dtype=jnp.int32)
    run_z, run_i = lax.fori_loop(0, climit_ref[i], body, (run_z0, run_i0))

    idx_ref[...] = run_i
    vis_ref[...] = (run_z < inf).astype(jnp.float32)


def _gather_body(coef_hbm, idx_hbm, g_hbm, idx_v, rows_v, sem):
    wid = lax.axis_index("s") * 2 + lax.axis_index("c")
    npix = (_H * _W) // 32
    nchunks = npix // _SC_CHUNK

    def chunk(ci, carry):
        base = wid * npix + ci * _SC_CHUNK
        pltpu.sync_copy(idx_hbm.at[pl.ds(base, _SC_CHUNK)], idx_v)
        pltpu.async_copy(coef_hbm.at[idx_v], rows_v, sem).wait()
        pltpu.sync_copy(rows_v, g_hbm.at[pl.ds(base, _SC_CHUNK)])
        return carry

    lax.fori_loop(0, nchunks, chunk, 0)


def _interp_body(g_ref, vis_ref, out_ref):
    pxf, pyf = _pix_coords(pl.program_id(0))
    g = g_ref[...]
    gp = g[:, 0:_D]
    gq = g[:, _D:2 * _D]
    gr = g[:, 2 * _D:3 * _D]
    vis = vis_ref[...]
    out8 = (gp * pyf + gq * pxf + gr) * vis
    out_ref[...] = jnp.concatenate(
        [out8, vis, jnp.zeros((_P, 16 - _D - 1), jnp.float32)], axis=1)


def kernel(vertices, faces, attributes):
    verts = vertices[0].astype(jnp.float32)
    f = faces[0]
    F = f.shape[0]

    fv = verts[f]
    x0, y0, z0 = fv[:, 0, 0], fv[:, 0, 1], fv[:, 0, 2]
    x1, y1, z1 = fv[:, 1, 0], fv[:, 1, 1], fv[:, 1, 2]
    x2, y2, z2 = fv[:, 2, 0], fv[:, 2, 1], fv[:, 2, 2]

    ymin = jnp.minimum(jnp.minimum(y0, y1), y2)
    order = jnp.argsort(ymin, stable=True)
    ymin_s = ymin[order]
    fvs = fv[order]
    x0, y0, z0 = fvs[:, 0, 0], fvs[:, 0, 1], fvs[:, 0, 2]
    x1, y1, z1 = fvs[:, 1, 0], fvs[:, 1, 1], fvs[:, 1, 2]
    x2, y2, z2 = fvs[:, 2, 0], fvs[:, 2, 1], fvs[:, 2, 2]

    area = (x1 - x0) * (y2 - y0) - (y1 - y0) * (x2 - x0)
    valid = jnp.abs(area) > 1e-8
    den = jnp.where(valid, area, 1.0)
    s = jnp.sign(den)

    e0x, e0y = x2 - x1, y2 - y1
    e1x, e1y = x0 - x2, y0 - y2
    c0 = e0y * x1 - e0x * y1
    c1 = e1y * x2 - e1x * y2

    na0, nb0 = s * e0x, -s * e0y
    na1, nb1 = s * e1x, -s * e1y
    nc0 = jnp.where(valid, s * c0, -1.0)
    na0 = jnp.where(valid, na0, 0.0)
    nb0 = jnp.where(valid, nb0, 0.0)
    nc1 = s * c1
    absden = jnp.abs(den)
    u0 = (z0 - z2) / absden
    u1 = (z1 - z2) / absden

    Fp = ((F + _FC - 1) // _FC) * _FC
    nchunks = Fp // _FC
    pad = Fp - F

    def padf(a):
        return jnp.pad(a, (0, pad))

    zero = jnp.zeros((Fp,), jnp.float32)
    fd = jnp.stack([
        padf(na0), padf(nb0), jnp.pad(nc0, (0, pad), constant_values=-1.0),
        padf(na1), padf(nb1), padf(nc1),
        padf(absden), padf(u0), padf(u1), padf(z2),
        zero, zero, zero, zero, zero, zero,
    ], axis=0)
    fd = fd.reshape(16, nchunks, _FC).transpose(1, 0, 2)

    nblocks = (_H * _W) // _P
    r0 = jnp.arange(nblocks, dtype=jnp.float32) * _ROWS_PER_BLOCK
    yhi = 1.0 - 2.0 * (r0 + 0.5) / _H
    counts = jnp.searchsorted(ymin_s, yhi, side='right')
    climit = ((counts + _FC - 1) // _FC).astype(jnp.int32)

    att = attributes[0].astype(jnp.float32)[order]
    e2x, e2y = x1 - x0, y1 - y0
    c2 = e2y * x0 - e2x * y0
    ex = jnp.stack([e0x, e1x, e2x], 1)
    ey = jnp.stack([e0y, e1y, e2y], 1)
    cc = jnp.stack([c0, c1, c2], 1)
    Pm = jnp.einsum('fk,fkd->fd', ex, att) / den[:, None]
    Qm = -jnp.einsum('fk,fkd->fd', ey, att) / den[:, None]
    Rm = jnp.einsum('fk,fkd->fd', cc, att) / den[:, None]
    coef = jnp.concatenate([Pm, Qm, Rm], axis=1)

    idx, vis = pl.pallas_call(
        _raster_body,
        grid_spec=pltpu.PrefetchScalarGridSpec(
            num_scalar_prefetch=1,
            grid=(nblocks,),
            in_specs=[
                pl.BlockSpec((nchunks, 16, _FC), lambda i, c: (0, 0, 0)),
            ],
            out_specs=[
                pl.BlockSpec((_P, 1), lambda i, c: (i, 0)),
                pl.BlockSpec((_P, 1), lambda i, c: (i, 0)),
            ],
        ),
        out_shape=[
            jax.ShapeDtypeStruct((_H * _W, 1), jnp.int32),
            jax.ShapeDtypeStruct((_H * _W, 1), jnp.float32),
        ],
    )(climit, fd)

    idx1 = idx.reshape(_H * _W)

    mesh = plsc.VectorSubcoreMesh(core_axis_name="c", subcore_axis_name="s")
    gathered = functools.partial(
        pl.kernel, mesh=mesh,
        out_type=jax.ShapeDtypeStruct((_H * _W, 3 * _D), jnp.float32),
        compiler_params=pltpu.CompilerParams(use_tc_tiling_on_sc=False),
        scratch_types=[
            pltpu.VMEM((_SC_CHUNK,), jnp.int32),
            pltpu.VMEM((_SC_CHUNK, 3 * _D), jnp.float32),
            pltpu.SemaphoreType.DMA,
        ],
    )(_gather_body)(coef, idx1)

    out = pl.pallas_call(
        _interp_body,
        grid=(nblocks,),
        in_specs=[
            pl.BlockSpec((_P, 3 * _D), lambda i: (i, 0)),
            pl.BlockSpec((_P, 1), lambda i: (i, 0)),
        ],
        out_specs=pl.BlockSpec((_P, 16), lambda i: (i, 0)),
        out_shape=jax.ShapeDtypeStruct((_H * _W, 16), jnp.float32),
    )(gathered, vis)

    img = out[:, 0:_D + 1].reshape(_H, _W, _D + 1).transpose(2, 0, 1)
    return img[None]
